# baseline (speedup 1.0000x reference)
import functools

import jax
import jax.numpy as jnp
from jax import lax
from jax.experimental import pallas as pl
from jax.experimental.pallas import tpu as pltpu
from jax.experimental.pallas import tpu_sc as plsc

N_EMBD = 64
SEQ_LEN = 2048
LANES = 16
IDX_MINOR = 128


def _tec_body(n_chunks, b_per_w, num_cores,
              table_hbm, idx_hbm, pos_hbm, out_hbm,
              idx_v, rows_v, pos_v, sem):
    wid = lax.axis_index("s") * num_cores + lax.axis_index("c")
    base = wid * b_per_w
    pbase = lax.rem(base, SEQ_LEN)

    pltpu.sync_copy(idx_hbm.at[pl.ds(wid * n_chunks, n_chunks)], idx_v)

    copies = []
    for c in range(n_chunks):
        copies.append(pltpu.async_copy(
            table_hbm.at[idx_v.at[c]],
            rows_v.at[pl.ds(c * IDX_MINOR, IDX_MINOR)],
            sem))
    pltpu.sync_copy(pos_hbm.at[pl.ds(pbase, b_per_w)], pos_v)
    for cp in copies:
        cp.wait()

    def add_row(i, _):
        for j in range(N_EMBD // LANES):
            sl = pl.ds(j * LANES, LANES)
            rows_v[i, sl] = rows_v[i, sl] + pos_v[i, sl]
        return 0

    lax.fori_loop(0, b_per_w, add_row, 0)

    pltpu.sync_copy(rows_v, out_hbm.at[pl.ds(base, b_per_w)])


def kernel(x, emb_table, pos_table):
    B, T = x.shape
    total = B * T
    info = plsc.get_sparse_core_info()
    num_workers = info.num_cores * info.num_subcores
    b_per_w = total // num_workers
    n_chunks = b_per_w // IDX_MINOR

    idx2d = x.reshape(total // IDX_MINOR, IDX_MINOR).astype(jnp.int32)

    mesh = plsc.VectorSubcoreMesh(core_axis_name="c", subcore_axis_name="s")
    body = functools.partial(_tec_body, n_chunks, b_per_w, info.num_cores)
    run = pl.kernel(
        body,
        mesh=mesh,
        compiler_params=pltpu.CompilerParams(use_tc_tiling_on_sc=False),
        out_type=jax.ShapeDtypeStruct((total, N_EMBD), jnp.float32),
        scratch_types=[
            pltpu.VMEM((n_chunks, IDX_MINOR), jnp.int32),
            pltpu.VMEM((b_per_w, N_EMBD), jnp.float32),
            pltpu.VMEM((b_per_w, N_EMBD), jnp.float32),
            pltpu.SemaphoreType.DMA,
        ],
    )
    out = run(emb_table, idx2d, pos_table)
    return out.reshape(B, T, N_EMBD)


# speedup vs baseline: 2.2952x; 2.2952x over previous
import functools

import jax
import jax.numpy as jnp
from jax import lax
from jax.experimental import pallas as pl
from jax.experimental.pallas import tpu as pltpu

N_EMBD = 64
SEQ_LEN = 2048
VOCAB = 1000000
K = 32


def _tc_body(*args):
    tiles_ref, lanes_ref = args[0], args[1]
    blocks = args[2:2 + K]
    pos_ref = args[2 + K]
    out_ref = args[3 + K]
    i = pl.program_id(0)
    iot = lax.broadcasted_iota(jnp.int32, (1, 128), 1)
    for j in range(K):
        lane = lanes_ref[i * K + j]
        oh = (iot == lane).astype(jnp.float32)
        col = lax.dot_general(oh, blocks[j][...],
                              dimension_numbers=(((1,), (1,)), ((), ())),
                              preferred_element_type=jnp.float32)
        out_ref[pl.ds(j, 1), :] = col + pos_ref[pl.ds(j, 1), :]


def kernel(x, emb_table, pos_table):
    B, T = x.shape
    total = B * T
    tT = emb_table.T
    xflat = x.reshape(total).astype(jnp.int32)
    tiles = xflat >> 7
    lanes = xflat & 127

    def tbl_map(j):
        def index_map(i, tiles_ref, lanes_ref):
            return (0, tiles_ref[i * K + j])
        return index_map

    def pos_map(i, tiles_ref, lanes_ref):
        return (i % (SEQ_LEN // K), 0)

    def out_map(i, tiles_ref, lanes_ref):
        return (i, 0)

    grid_spec = pltpu.PrefetchScalarGridSpec(
        num_scalar_prefetch=2,
        grid=(total // K,),
        in_specs=[pl.BlockSpec((N_EMBD, 128), tbl_map(j)) for j in range(K)]
        + [pl.BlockSpec((K, N_EMBD), pos_map)],
        out_specs=pl.BlockSpec((K, N_EMBD), out_map),
    )
    out = pl.pallas_call(
        _tc_body,
        grid_spec=grid_spec,
        out_shape=jax.ShapeDtypeStruct((total, N_EMBD), jnp.float32),
        compiler_params=pltpu.CompilerParams(
            dimension_semantics=("arbitrary",)),
    )(tiles, lanes, *([tT] * K), pos_table)
    return out.reshape(B, T, N_EMBD)
